# baseline (speedup 1.0000x reference)
import jax
import jax.numpy as jnp
from jax.experimental import pallas as pl
from jax.experimental.pallas import tpu as pltpu

_B = 4096
_C = 1000
_NS = 8
_R = _B // _NS


def _rce_kernel(*refs):
    x_refs = refs[:_NS]
    y_ref = refs[_NS]
    out_ref = refs[_NS + 1]

    colsum = jnp.zeros((1, _C), jnp.float32)
    for r in x_refs:
        colsum = colsum + jnp.sum(r[0], axis=0, keepdims=True)

    classes = jax.lax.broadcasted_iota(jnp.int32, (1, _C), 1)
    counts = jnp.zeros((1, _C), jnp.float32)
    for k in range(_NS):
        yv = y_ref[k].reshape(_R, 1)
        counts = counts + jnp.sum((yv == classes).astype(jnp.float32),
                                  axis=0, keepdims=True)

    s = jnp.sum(colsum * counts, keepdims=True)
    out_ref[...] = 6.0 - (6.0 / (_B * _B)) * s


def kernel(x, y):
    x3 = x.reshape(_NS, _R, _C)
    y3 = y.astype(jnp.int32).reshape(_NS, 1, _R)
    in_specs = [
        pl.BlockSpec((1, _R, _C), lambda i, k=k: (k, 0, 0)) for k in range(_NS)
    ]
    in_specs.append(pl.BlockSpec((_NS, 1, _R), lambda i: (0, 0, 0)))
    out = pl.pallas_call(
        _rce_kernel,
        grid=(1,),
        in_specs=in_specs,
        out_specs=pl.BlockSpec((1, 1), lambda i: (0, 0)),
        out_shape=jax.ShapeDtypeStruct((1, 1), jnp.float32),
    )(*([x3] * _NS), y3)
    return jnp.reshape(out, ())


# speedup vs baseline: 1.0468x; 1.0468x over previous
import jax
import jax.numpy as jnp
from jax.experimental import pallas as pl
from jax.experimental.pallas import tpu as pltpu

_B = 4096
_C = 1000
_NS = 8
_R = _B // _NS


def _rce_kernel(x_hbm, y_ref, out_ref, buf, sems):
    copies = [
        pltpu.make_async_copy(x_hbm.at[k], buf.at[k], sems.at[k])
        for k in range(_NS)
    ]
    for c in copies:
        c.start()

    classes = jax.lax.broadcasted_iota(jnp.int32, (1, _C), 1)
    counts = jnp.zeros((1, _C), jnp.float32)
    for k in range(_NS):
        yv = y_ref[k].reshape(_R, 1)
        counts = counts + jnp.sum((yv == classes).astype(jnp.float32),
                                  axis=0, keepdims=True)

    colsum = jnp.zeros((1, _C), jnp.float32)
    for k in range(_NS):
        copies[k].wait()
        colsum = colsum + jnp.sum(buf[k], axis=0, keepdims=True)

    s = jnp.sum(colsum * counts, keepdims=True)
    out_ref[...] = 6.0 - (6.0 / (_B * _B)) * s


def kernel(x, y):
    x3 = x.reshape(_NS, _R, _C)
    y3 = y.astype(jnp.int32).reshape(_NS, 1, _R)
    out = pl.pallas_call(
        _rce_kernel,
        in_specs=[
            pl.BlockSpec(memory_space=pl.ANY),
            pl.BlockSpec((_NS, 1, _R), lambda: (0, 0, 0)),
        ],
        out_specs=pl.BlockSpec((1, 1), lambda: (0, 0)),
        out_shape=jax.ShapeDtypeStruct((1, 1), jnp.float32),
        scratch_shapes=[
            pltpu.VMEM((_NS, _R, _C), jnp.float32),
            pltpu.SemaphoreType.DMA((_NS,)),
        ],
    )(x3, y3)
    return jnp.reshape(out, ())


# speedup vs baseline: 1.7691x; 1.6900x over previous
import jax
import jax.numpy as jnp
from jax.experimental import pallas as pl
from jax.experimental.pallas import tpu as pltpu

_B = 4096
_C = 1000
_NS = 8
_R = _B // _NS


def _rce_kernel(x_hbm, y_ref, out_ref, buf, sems):
    copies = [
        pltpu.make_async_copy(x_hbm.at[pl.ds(k * _R, _R)], buf.at[k], sems.at[k])
        for k in range(_NS)
    ]
    for c in copies:
        c.start()

    classes = jax.lax.broadcasted_iota(jnp.int32, (1, _C), 1)
    counts = jnp.zeros((1, _C), jnp.float32)
    for k in range(_NS):
        yv = y_ref[k].reshape(_R, 1)
        counts = counts + jnp.sum((yv == classes).astype(jnp.float32),
                                  axis=0, keepdims=True)

    colsum = jnp.zeros((1, _C), jnp.float32)
    for k in range(_NS):
        copies[k].wait()
        colsum = colsum + jnp.sum(buf[k], axis=0, keepdims=True)

    s = jnp.sum(colsum * counts, keepdims=True)
    out_ref[...] = 6.0 - (6.0 / (_B * _B)) * s


def kernel(x, y):
    y3 = y.astype(jnp.int32).reshape(_NS, 1, _R)
    out = pl.pallas_call(
        _rce_kernel,
        in_specs=[
            pl.BlockSpec(memory_space=pl.ANY),
            pl.BlockSpec((_NS, 1, _R), lambda: (0, 0, 0)),
        ],
        out_specs=pl.BlockSpec((1, 1), lambda: (0, 0)),
        out_shape=jax.ShapeDtypeStruct((1, 1), jnp.float32),
        scratch_shapes=[
            pltpu.VMEM((_NS, _R, _C), jnp.float32),
            pltpu.SemaphoreType.DMA((_NS,)),
        ],
    )(x, y3)
    return jnp.reshape(out, ())
